# baseline (speedup 1.0000x reference)
import functools
import math

import jax
import jax.numpy as jnp
from jax import lax
from jax.experimental import pallas as pl
from jax.experimental.pallas import tpu as pltpu
from jax.experimental.pallas import tpu_sc as plsc

_EMB_DIM = 64
_SCALE = math.sqrt(_EMB_DIM)
_LANES = 16
_CHUNK = 128


@functools.lru_cache(maxsize=None)
def _build(vocab: int, dim: int, n: int):
    info = plsc.get_sparse_core_info()
    nc, ns = info.num_cores, info.num_subcores
    nw = nc * ns
    assert n % (nw * _CHUNK) == 0
    per_w = n // nw
    chunks = per_w // _CHUNK
    vecs_per_row = dim // _LANES

    mesh = plsc.VectorSubcoreMesh(core_axis_name="c", subcore_axis_name="s")

    @functools.partial(
        pl.kernel,
        mesh=mesh,
        out_type=jax.ShapeDtypeStruct((n, dim), jnp.float32),
        scratch_types=[
            pltpu.VMEM((_CHUNK,), jnp.int32),
            pltpu.VMEM((_CHUNK, dim), jnp.float32),
            pltpu.SemaphoreType.DMA,
        ],
        compiler_params=pltpu.CompilerParams(use_tc_tiling_on_sc=False),
    )
    def emb_kernel(table_hbm, idx_hbm, out_hbm, idx_v, rows_v, sem):
        wid = lax.axis_index("s") * nc + lax.axis_index("c")
        base = wid * per_w

        def body(g, carry):
            off = base + g * _CHUNK
            pltpu.sync_copy(idx_hbm.at[pl.ds(off, _CHUNK)], idx_v)
            pltpu.async_copy(table_hbm.at[idx_v], rows_v, sem).wait()

            def scale_row(r, c2):
                for c in range(vecs_per_row):
                    sl = pl.ds(c * _LANES, _LANES)
                    rows_v[r, sl] = rows_v[r, sl] * _SCALE
                return c2

            lax.fori_loop(0, _CHUNK, scale_row, 0)
            pltpu.sync_copy(rows_v, out_hbm.at[pl.ds(off, _CHUNK)])
            return carry

        lax.fori_loop(0, chunks, body, 0)

    return emb_kernel


def kernel(persona, table):
    b, h = persona.shape
    vocab, dim = table.shape
    idx = persona.reshape(-1).astype(jnp.int32)
    out = _build(vocab, dim, b * h)(table, idx)
    return out.reshape(b, h, dim)


# speedup vs baseline: 1.2675x; 1.2675x over previous
import functools
import math

import jax
import jax.numpy as jnp
from jax import lax
from jax.experimental import pallas as pl
from jax.experimental.pallas import tpu as pltpu
from jax.experimental.pallas import tpu_sc as plsc

_EMB_DIM = 64
_SCALE = math.sqrt(_EMB_DIM)
_LANES = 16
_CHUNK = 128
_NBUF = 4
_ROWS_PER_IT = 4


@functools.lru_cache(maxsize=None)
def _build(vocab: int, dim: int, n: int):
    info = plsc.get_sparse_core_info()
    nc, ns = info.num_cores, info.num_subcores
    nw = nc * ns
    assert n % (nw * _CHUNK) == 0
    per_w = n // nw
    chunks = per_w // _CHUNK
    groups = chunks // _NBUF
    assert chunks % _NBUF == 0 and groups >= 2
    vecs = dim // _LANES

    mesh = plsc.VectorSubcoreMesh(core_axis_name="c", subcore_axis_name="s")

    @functools.partial(
        pl.kernel,
        mesh=mesh,
        out_type=jax.ShapeDtypeStruct((n, dim), jnp.float32),
        scratch_types=[
            pltpu.VMEM((chunks, _CHUNK), jnp.int32),
            pltpu.VMEM((_NBUF, _CHUNK, dim), jnp.float32),
            pltpu.VMEM((_NBUF, _CHUNK, dim), jnp.float32),
            pltpu.SemaphoreType.DMA((_NBUF,)),
            pltpu.SemaphoreType.DMA((_NBUF,)),
        ],
        compiler_params=pltpu.CompilerParams(use_tc_tiling_on_sc=False),
    )
    def emb_kernel(table_hbm, idx_hbm, out_hbm, idx_v, bin_v, bout_v,
                   sem_in, sem_out):
        wid = lax.axis_index("s") * nc + lax.axis_index("c")
        base = wid * per_w
        pltpu.sync_copy(idx_hbm.at[wid], idx_v)

        def start_gather(g, b):
            pltpu.async_copy(table_hbm.at[idx_v.at[g]], bin_v.at[b],
                             sem_in.at[b])

        def wait_gather(b):
            pltpu.make_async_copy(table_hbm.at[idx_v.at[0]], bin_v.at[b],
                                  sem_in.at[b]).wait()

        def start_wb(g, b):
            pltpu.async_copy(bout_v.at[b],
                             out_hbm.at[pl.ds(base + g * _CHUNK, _CHUNK)],
                             sem_out.at[b])

        def wait_wb(b):
            pltpu.make_async_copy(bout_v.at[b],
                                  out_hbm.at[pl.ds(base, _CHUNK)],
                                  sem_out.at[b]).wait()

        def scale(b):
            def sbody(i, c):
                for dr in range(_ROWS_PER_IT):
                    r = i * _ROWS_PER_IT + dr
                    for c4 in range(vecs):
                        sl = pl.ds(c4 * _LANES, _LANES)
                        bout_v[b, r, sl] = bin_v[b, r, sl] * _SCALE
                return c
            lax.fori_loop(0, _CHUNK // _ROWS_PER_IT, sbody, 0)

        for b in range(_NBUF):
            start_gather(b, b)
        for b in range(_NBUF):
            wait_gather(b)
            scale(b)
            start_gather(_NBUF + b, b)
            start_wb(b, b)

        def mbody(g0, c):
            for b in range(_NBUF):
                g = g0 * _NBUF + b
                wait_gather(b)
                wait_wb(b)
                scale(b)
                start_gather(g + _NBUF, b)
                start_wb(g, b)
            return c

        lax.fori_loop(1, groups - 1, mbody, 0)

        for b in range(_NBUF):
            g = (groups - 1) * _NBUF + b
            wait_gather(b)
            wait_wb(b)
            scale(b)
            start_wb(g, b)
        for b in range(_NBUF):
            wait_wb(b)

    return emb_kernel


def kernel(persona, table):
    b, h = persona.shape
    vocab, dim = table.shape
    n = b * h
    info = plsc.get_sparse_core_info()
    nw = info.num_cores * info.num_subcores
    idx = persona.reshape(nw, n // (nw * _CHUNK), _CHUNK).astype(jnp.int32)
    out = _build(vocab, dim, n)(table, idx)
    return out.reshape(b, h, dim)
